# baseline (speedup 1.0000x reference)
import functools

import jax
import jax.numpy as jnp
from jax import lax
from jax.experimental import pallas as pl
from jax.experimental.pallas import tpu as pltpu
from jax.experimental.pallas import tpu_sc as plsc

_D = 32
_B = 16384
_CHUNK = 128
_L = 16

_info = plsc.get_sparse_core_info()
_NC, _NS = _info.num_cores, _info.num_subcores
_NW = _NC * _NS
_B_PER_W = _B // _NW
_N_CHUNK = _B_PER_W // _CHUNK
_N_GROUP = _B_PER_W // _L

_mesh = plsc.VectorSubcoreMesh(core_axis_name="c", subcore_axis_name="s")


@functools.partial(
    pl.kernel,
    mesh=_mesh,
    compiler_params=pltpu.CompilerParams(needs_layout_passes=False),
    out_type=jax.ShapeDtypeStruct((_B * _D // _CHUNK, _CHUNK), jnp.float32),
    scratch_types=[
        pltpu.VMEM((_N_CHUNK, _CHUNK), jnp.int32),
        pltpu.VMEM((_N_CHUNK, _CHUNK), jnp.int32),
        pltpu.VMEM((_B_PER_W,), jnp.int32),
        pltpu.VMEM((_B_PER_W, 4 * _D), jnp.float32),
        pltpu.VMEM((_B_PER_W * _D // _CHUNK, _CHUNK), jnp.float32),
        pltpu.SemaphoreType.DMA,
    ],
)
def _embed(idx_hbm, table_hbm, out_hbm, idx_v, idx4_v, cb_v, buf_v, out_v, sem):
    wid = lax.axis_index("s") * _NC + lax.axis_index("c")
    base = wid * _N_CHUNK
    pltpu.sync_copy(idx_hbm.at[pl.ds(base, _N_CHUNK)], idx_v)

    for k in range(_B_PER_W // _L):
        r, c = k // (_CHUNK // _L), (k % (_CHUNK // _L)) * _L
        t = idx_v[r, pl.ds(c, _L)]
        idx4_v[r, pl.ds(c, _L)] = lax.shift_right_logical(t, 2)
        cb_v[pl.ds(k * _L, _L)] = lax.shift_left(t & 3, 5)

    copies = [
        pltpu.async_copy(
            table_hbm.at[idx4_v.at[j]], buf_v.at[pl.ds(j * _CHUNK, _CHUNK)], sem
        )
        for j in range(_N_CHUNK)
    ]
    for c in copies:
        c.wait()

    def select(g, _):
        iv = g * _L + lax.iota(jnp.int32, _L)
        cb = cb_v[pl.ds(g * _L, _L)]
        f0 = iv * _D
        for j in range(_D):
            vals = plsc.load_gather(buf_v, [iv, cb + j])
            flat = f0 + j
            plsc.store_scatter(
                out_v, [lax.shift_right_logical(flat, 7), flat & (_CHUNK - 1)], vals
            )
        return 0

    lax.fori_loop(0, _N_GROUP, select, 0)
    n_out_rows = _B_PER_W * _D // _CHUNK
    pltpu.sync_copy(out_v, out_hbm.at[pl.ds(wid * n_out_rows, n_out_rows)])


def kernel(x, weight):
    idx = x.astype(jnp.int32).reshape(_B // _CHUNK, _CHUNK)
    table = weight.reshape(250000, 4 * _D)
    return _embed(idx, table).reshape(_B, _D)


# speedup vs baseline: 1.0309x; 1.0309x over previous
import functools

import jax
import jax.numpy as jnp
from jax import lax
from jax.experimental import pallas as pl
from jax.experimental.pallas import tpu as pltpu
from jax.experimental.pallas import tpu_sc as plsc

_D = 32
_B = 16384
_CHUNK = 128
_L = 16

_info = plsc.get_sparse_core_info()
_NC, _NS = _info.num_cores, _info.num_subcores
_NW = _NC * _NS
_B_PER_W = _B // _NW
_N_CHUNK = _B_PER_W // _CHUNK
_N_GROUP = _B_PER_W // _L

_mesh = plsc.VectorSubcoreMesh(core_axis_name="c", subcore_axis_name="s")


@functools.partial(
    pl.kernel,
    mesh=_mesh,
    compiler_params=pltpu.CompilerParams(needs_layout_passes=False),
    out_type=jax.ShapeDtypeStruct((_D, _B), jnp.float32),
    scratch_types=[
        pltpu.VMEM((_N_CHUNK, _CHUNK), jnp.int32),
        pltpu.VMEM((_N_CHUNK, _CHUNK), jnp.int32),
        pltpu.VMEM((_B_PER_W,), jnp.int32),
        pltpu.VMEM((_B_PER_W, 4 * _D), jnp.float32),
        pltpu.VMEM((_D, _B_PER_W), jnp.float32),
        pltpu.SemaphoreType.DMA,
    ],
)
def _embed(idx_hbm, table_hbm, out_hbm, idx_v, idx4_v, cb_v, buf_v, out_v, sem):
    wid = lax.axis_index("s") * _NC + lax.axis_index("c")
    base = wid * _N_CHUNK
    pltpu.sync_copy(idx_hbm.at[pl.ds(base, _N_CHUNK)], idx_v)

    for k in range(_B_PER_W // _L):
        r, c = k // (_CHUNK // _L), (k % (_CHUNK // _L)) * _L
        t = idx_v[r, pl.ds(c, _L)]
        idx4_v[r, pl.ds(c, _L)] = lax.shift_right_logical(t, 2)
        cb_v[pl.ds(k * _L, _L)] = lax.shift_left(t & 3, 5)

    copies = [
        pltpu.async_copy(
            table_hbm.at[idx4_v.at[j]], buf_v.at[pl.ds(j * _CHUNK, _CHUNK)], sem
        )
        for j in range(_N_CHUNK)
    ]
    for cp in copies:
        cp.wait()

    for g in range(_N_GROUP):
        iv = g * _L + lax.iota(jnp.int32, _L)
        cb = cb_v[pl.ds(g * _L, _L)]
        for j in range(_D):
            vals = plsc.load_gather(buf_v, [iv, cb + j])
            plsc.store_scatter(out_v, [jnp.full((_L,), j, jnp.int32), iv], vals)

    pltpu.sync_copy(out_v, out_hbm.at[:, pl.ds(wid * _B_PER_W, _B_PER_W)])


def kernel(x, weight):
    idx = x.astype(jnp.int32).reshape(_B // _CHUNK, _CHUNK)
    table = weight.reshape(250000, 4 * _D)
    return _embed(idx, table).T
